# Initial kernel scaffold; baseline (speedup 1.0000x reference)
import jax
import jax.numpy as jnp
from jax.experimental import pallas as pl


def kernel(features):
    raise NotImplementedError("write your pallas kernel here")


# speedup vs baseline: 34.1284x; 34.1284x over previous
import functools

import jax
import jax.numpy as jnp
import numpy as np
from jax.experimental import pallas as pl

_K = 256
_SIGN = np.int32(np.uint32(0x80000000))


def _topk_mask_block(x_ref, o_ref, *, k):
    x = x_ref[...]
    b = jax.lax.bitcast_convert_type(x, jnp.int32)
    s = jnp.where(b >= 0, b, b ^ np.int32(0x7FFFFFFF))
    rows = x.shape[0]
    p = jnp.zeros((rows, 1), jnp.int32)
    for i in range(31, -1, -1):
        bit = np.int32(np.uint32(1 << i))
        trial = p | bit
        thr = trial ^ _SIGN
        cnt = jnp.sum((s >= thr).astype(jnp.int32), axis=1, keepdims=True)
        p = jnp.where(cnt >= k, trial, p)
    tau = p ^ _SIGN
    o_ref[...] = jnp.where(s >= tau, jnp.maximum(x, 0.0), 0.0)


@jax.jit
def kernel(features):
    batch, layers, d = features.shape
    rows = batch * layers
    x = features.reshape(rows, d)
    block_rows = 64 if rows % 64 == 0 else rows
    out = pl.pallas_call(
        functools.partial(_topk_mask_block, k=_K),
        grid=(rows // block_rows,),
        in_specs=[pl.BlockSpec((block_rows, d), lambda i: (i, 0))],
        out_specs=pl.BlockSpec((block_rows, d), lambda i: (i, 0)),
        out_shape=jax.ShapeDtypeStruct((rows, d), features.dtype),
    )(x)
    return out.reshape(batch, layers, d)


# speedup vs baseline: 49.6864x; 1.4559x over previous
import functools

import jax
import jax.numpy as jnp
import numpy as np
from jax.experimental import pallas as pl

_K = 256
_SIGN = np.int32(np.uint32(0x80000000))


def _count16(m):
    rows, d = m.shape
    chunks = 64
    w = d // chunks
    acc = m[:, :w]
    for j in range(1, chunks):
        acc = acc + m[:, j * w:(j + 1) * w]
    return jnp.sum(acc.astype(jnp.int32), axis=1, keepdims=True)


def _topk_mask_block(x_ref, o_ref, *, k):
    x = x_ref[...]
    rows = x.shape[0]
    b = jax.lax.bitcast_convert_type(x, jnp.int32)
    s = jnp.where(b >= 0, b, b ^ np.int32(0x7FFFFFFF))
    hi = (s >> 16).astype(jnp.int16)
    lob = (s ^ np.int32(0x8000)).astype(jnp.int16)

    one16 = jnp.int16(1)
    zero16 = jnp.int16(0)

    p = jnp.zeros((rows, 1), jnp.int32)
    for i in range(15, -1, -1):
        trial = p | np.int32(1 << i)
        thr = (trial ^ np.int32(0x8000)).astype(jnp.int16)
        cnt = _count16(jnp.where(hi >= thr, one16, zero16))
        p = jnp.where(cnt >= k, trial, p)

    h = (p ^ np.int32(0x8000)).astype(jnp.int16)
    c_hi = _count16(jnp.where(hi > h, one16, zero16))
    q = jnp.where(hi == h, lob, jnp.int16(-32768))
    r = k - c_hi

    p2 = jnp.zeros((rows, 1), jnp.int32)
    for i in range(15, -1, -1):
        trial = p2 | np.int32(1 << i)
        thr = (trial ^ np.int32(0x8000)).astype(jnp.int16)
        cnt = _count16(jnp.where(q >= thr, one16, zero16))
        p2 = jnp.where(cnt >= r, trial, p2)

    pu = (p << 16) | p2
    tf_bits = jnp.where(pu < 0, pu ^ _SIGN, ~pu)
    tf = jax.lax.bitcast_convert_type(tf_bits, jnp.float32)
    o_ref[...] = jnp.where(x >= tf, jnp.maximum(x, 0.0), 0.0)


@jax.jit
def kernel(features):
    batch, layers, d = features.shape
    rows = batch * layers
    x = features.reshape(rows, d)
    block_rows = 64 if rows % 64 == 0 else rows
    out = pl.pallas_call(
        functools.partial(_topk_mask_block, k=_K),
        grid=(rows // block_rows,),
        in_specs=[pl.BlockSpec((block_rows, d), lambda i: (i, 0))],
        out_specs=pl.BlockSpec((block_rows, d), lambda i: (i, 0)),
        out_shape=jax.ShapeDtypeStruct((rows, d), features.dtype),
    )(x)
    return out.reshape(batch, layers, d)
